# baseline (speedup 1.0000x reference)
import functools

import jax
import jax.numpy as jnp
from jax import lax
from jax.experimental import pallas as pl
from jax.experimental.pallas import tpu as pltpu
from jax.experimental.pallas import tpu_sc as plsc

_L = 16


def _allreduce_sum(v):
    lanes = lax.iota(jnp.int32, _L)
    dnums = lax.GatherDimensionNumbers(offset_dims=(), collapsed_slice_dims=(0,),
                                       start_index_map=(0,))
    for k in (8, 4, 2, 1):
        idx = jnp.bitwise_xor(lanes, jnp.full((_L,), k, jnp.int32))
        v = v + lax.gather(v, idx[:, None], dnums, slice_sizes=(1,),
                           mode=lax.GatherScatterMode.PROMISE_IN_BOUNDS)
    return v


def _rsqrt_vec(x):
    i = lax.bitcast_convert_type(x, jnp.int32)
    i = jnp.int32(0x5F3759DF) - lax.shift_right_arithmetic(i, jnp.int32(1))
    y = lax.bitcast_convert_type(i, jnp.float32)
    for _ in range(3):
        y = y * (jnp.float32(1.5) - jnp.float32(0.5) * x * y * y)
    return y


def _make_kernel(B, S, H, C, eps):
    NCH = S // C
    assert NCH % 2 == 0
    mesh = plsc.VectorSubcoreMesh(core_axis_name="c", subcore_axis_name="s")
    H2 = H // 2
    HJ2 = H // (2 * _L)
    NIDX = 8 * C
    inv_h = jnp.float32(1.0 / H)

    def body(idxs_hbm, idxw_hbm, word_hbm, cat_hbm, gamma_hbm, beta_hbm,
             out_hbm,
             idxs_v, idxw_v, word_v, small_v, acc_v, stat_v, gam_v, bet_v,
             sem0, sem1):
        cid = lax.axis_index("c")
        sid = lax.axis_index("s")
        wid = sid * 2 + cid

        pltpu.sync_copy(idxs_hbm.at[wid], idxs_v)
        pltpu.sync_copy(idxw_hbm.at[wid], idxw_v)
        pltpu.sync_copy(gamma_hbm, gam_v)
        pltpu.sync_copy(beta_hbm, bet_v)

        def copies(c, slot, sem, mk):
            return [
                mk(cat_hbm.at[idxs_v.at[c]], small_v.at[slot], sem),
                mk(word_hbm.at[idxw_v.at[pl.ds(c * C, C)]],
                   word_v.at[slot], sem),
            ]

        def issue(c, slot, sem):
            copies(c, slot, sem, pltpu.async_copy)

        def drain(c, slot, sem):
            for cp in copies(c, slot, sem, pltpu.make_async_copy):
                cp.wait()

        def compute(c, slot):
            zero = jnp.zeros((_L,), jnp.float32)
            for t in range(C):
                stat_v[0, t, :] = zero
                stat_v[1, t, :] = zero

            sixteen = jnp.full((_L,), 16, jnp.int32)

            def _tree(vs):
                while len(vs) > 1:
                    nxt = [vs[i] + vs[i + 1]
                           for i in range(0, len(vs) - 1, 2)]
                    if len(vs) % 2:
                        nxt.append(vs[-1])
                    vs = nxt
                return vs[0]

            def j_body(jj):
                basew = pl.multiple_of(jj * _L, _L)
                colw = pl.ds(basew, _L)
                base = pl.multiple_of(2 * jj * _L, 2 * _L)
                c0 = pl.ds(base, _L)
                c1 = pl.ds(base + _L, _L)
                for t in range(C):
                    xs = [small_v[slot, k * C + t, colw] for k in range(8)]
                    lo = [lax.bitcast_convert_type(
                              lax.shift_left(x, sixteen), jnp.float32)
                          for x in xs]
                    hi = [lax.bitcast_convert_type(x, jnp.float32)
                          for x in xs]
                    a0 = _tree(lo + [word_v[slot, t, c0]])
                    a1 = _tree(hi + [word_v[slot, t, c1]])
                    acc_v[t, c0] = a0
                    acc_v[t, c1] = a1
                    plsc.addupdate(stat_v.at[0, t], a0 + a1)
                    plsc.addupdate(stat_v.at[1, t], a0 * a0 + a1 * a1)

            plsc.parallel_loop(0, HJ2, unroll=2)(j_body)
            mus = []
            rs = []
            for t in range(C):
                s = _allreduce_sum(stat_v[0, t])
                q = _allreduce_sum(stat_v[1, t])
                mu = s * inv_h
                var = q * inv_h - mu * mu
                mus.append(mu)
                rs.append(_rsqrt_vec(var + jnp.float32(eps)))

            def j2_body(jj):
                col = pl.ds(jj * _L, _L)
                g = gam_v[col]
                b = bet_v[col]
                for t in range(C):
                    acc_v[t, col] = (acc_v[t, col] - mus[t]) * rs[t] * g + b

            plsc.parallel_loop(0, H // _L, unroll=2)(j2_body)
            pltpu.sync_copy(acc_v, out_hbm.at[wid, pl.ds(c * C, C)])

        issue(0, 0, sem0)

        def pair_body(p, carry):
            c0 = p * 2
            c1 = c0 + 1
            issue(c1, 1, sem1)
            drain(c0, 0, sem0)
            compute(c0, 0)

            @pl.when(p < NCH // 2 - 1)
            def _():
                issue(c0 + 2, 0, sem0)

            drain(c1, 1, sem1)
            compute(c1, 1)
            return carry

        lax.fori_loop(0, NCH // 2, pair_body, 0)

    return pl.kernel(
        body,
        out_type=jax.ShapeDtypeStruct((B, S, H), jnp.float32),
        mesh=mesh,
        scratch_types=[
            pltpu.VMEM((NCH, NIDX), jnp.int32),
            pltpu.VMEM((S,), jnp.int32),
            pltpu.VMEM((2, C, H), jnp.float32),
            pltpu.VMEM((2, NIDX, H2), jnp.int32),
            pltpu.VMEM((C, H), jnp.float32),
            pltpu.VMEM((2, C, _L), jnp.float32),
            pltpu.VMEM((H,), jnp.float32),
            pltpu.VMEM((H,), jnp.float32),
            pltpu.SemaphoreType.DMA,
            pltpu.SemaphoreType.DMA,
        ],
    )


def _to_bf16_perm(t):
    v, h = t.shape
    b = t.astype(jnp.bfloat16)
    b = b.reshape(v, h // 32, 2, 16).transpose(0, 1, 3, 2)
    return lax.bitcast_convert_type(b, jnp.int32).reshape(v, h // 2)


def kernel(input_ids, bbox, token_type_ids, word_emb, x_emb, y_emb, h_emb,
           w_emb, pos_emb, tok_emb, gamma, beta):
    B, S = input_ids.shape
    H = word_emb.shape[1]
    C = 8
    nx = x_emb.shape[0]
    ny = y_emb.shape[0]
    nh = h_emb.shape[0]
    nw = w_emb.shape[0]
    nt = tok_emb.shape[0]
    cat = jnp.concatenate([
        _to_bf16_perm(x_emb), _to_bf16_perm(y_emb), _to_bf16_perm(h_emb),
        _to_bf16_perm(w_emb), _to_bf16_perm(tok_emb),
        _to_bf16_perm(pos_emb)], axis=0)
    off_y = nx
    off_h = nx + ny
    off_w = off_h + nh
    off_t = off_w + nw
    off_p = off_t + nt
    b0 = bbox[:, :, 0]
    b1 = bbox[:, :, 1]
    b2 = bbox[:, :, 2]
    b3 = bbox[:, :, 3]
    pos_ids = jnp.broadcast_to(jnp.arange(S, dtype=jnp.int32)[None, :],
                               (B, S))
    idxs = jnp.stack([
        b0, b1 + off_y, b2, b3 + off_y, (b3 - b1) + off_h,
        (b2 - b0) + off_w, token_type_ids.astype(jnp.int32) + off_t,
        pos_ids + off_p], axis=1)
    idxs = (idxs.reshape(B, 8, S // C, C).transpose(0, 2, 1, 3)
            .reshape(B, S // C, 8 * C))
    k = _make_kernel(B, S, H, C, 1e-05)
    return k(idxs, input_ids.astype(jnp.int32), word_emb, cat, gamma,
             beta)


# speedup vs baseline: 2.2109x; 2.2109x over previous
import functools

import jax
import jax.numpy as jnp
from jax import lax
from jax.experimental import pallas as pl
from jax.experimental.pallas import tpu as pltpu
from jax.experimental.pallas import tpu_sc as plsc

_L = 16


def _allreduce_sum(v):
    lanes = lax.iota(jnp.int32, _L)
    dnums = lax.GatherDimensionNumbers(offset_dims=(), collapsed_slice_dims=(0,),
                                       start_index_map=(0,))
    for k in (8, 4, 2, 1):
        idx = jnp.bitwise_xor(lanes, jnp.full((_L,), k, jnp.int32))
        v = v + lax.gather(v, idx[:, None], dnums, slice_sizes=(1,),
                           mode=lax.GatherScatterMode.PROMISE_IN_BOUNDS)
    return v


def _rsqrt_vec(x):
    i = lax.bitcast_convert_type(x, jnp.int32)
    i = jnp.int32(0x5F3759DF) - lax.shift_right_arithmetic(i, jnp.int32(1))
    y = lax.bitcast_convert_type(i, jnp.float32)
    for _ in range(3):
        y = y * (jnp.float32(1.5) - jnp.float32(0.5) * x * y * y)
    return y


def _make_kernel(B, S, H, C, eps):
    NCH = S // C
    DEPTH = 4
    assert NCH % DEPTH == 0
    mesh = plsc.VectorSubcoreMesh(core_axis_name="c", subcore_axis_name="s")
    H2 = H // 2
    HJ2 = H // (2 * _L)
    inv_h = jnp.float32(1.0 / H)

    def body(idx_hbm, word_hbm, x_hbm, y_hbm, h_hbm, w_hbm, pt_hbm,
             gamma_hbm, beta_hbm, out_hbm,
             idx_v, word_v, small_v, stat_v, gam_v, bet_v,
             sem0, sem1, sem2, sem3):
        sems = (sem0, sem1, sem2, sem3)
        cid = lax.axis_index("c")
        sid = lax.axis_index("s")
        wid = sid * 2 + cid

        pltpu.sync_copy(idx_hbm.at[wid], idx_v)
        pltpu.sync_copy(gamma_hbm, gam_v)
        pltpu.sync_copy(beta_hbm, bet_v)

        smalls = (x_hbm, y_hbm, x_hbm, y_hbm, h_hbm, w_hbm, pt_hbm)

        def copies(c, slot, mk):
            sem = sems[slot]
            cs = [mk(word_hbm.at[idx_v.at[0, pl.ds(c * C, C)]],
                     word_v.at[slot], sem)]
            cs += [mk(tab.at[idx_v.at[k + 1, pl.ds(c * C, C)]],
                      small_v.at[slot, k], sem)
                   for k, tab in enumerate(smalls)]
            return cs

        def issue(c, slot):
            copies(c, slot, pltpu.async_copy)

        def drain(c, slot):
            for cp in copies(c, slot, pltpu.make_async_copy):
                cp.wait()

        def compute(c, slot):
            zero = jnp.zeros((_L,), jnp.float32)
            for t in range(C):
                stat_v[0, t, :] = zero
                stat_v[1, t, :] = zero

            sixteen = jnp.full((_L,), 16, jnp.int32)

            def _tree(vs):
                while len(vs) > 1:
                    nxt = [vs[i] + vs[i + 1]
                           for i in range(0, len(vs) - 1, 2)]
                    if len(vs) % 2:
                        nxt.append(vs[-1])
                    vs = nxt
                return vs[0]

            def j_body(jj):
                basew = pl.multiple_of(jj * _L, _L)
                colw = pl.ds(basew, _L)
                base = pl.multiple_of(2 * jj * _L, 2 * _L)
                c0 = pl.ds(base, _L)
                c1 = pl.ds(base + _L, _L)
                for t in range(C):
                    xs = [small_v[slot, k, t, colw] for k in range(7)]
                    lo = [lax.bitcast_convert_type(
                              lax.shift_left(x, sixteen), jnp.float32)
                          for x in xs]
                    hi = [lax.bitcast_convert_type(x, jnp.float32)
                          for x in xs]
                    a0 = _tree(lo + [word_v[slot, t, c0]])
                    a1 = _tree(hi + [word_v[slot, t, c1]])
                    word_v[slot, t, c0] = a0
                    word_v[slot, t, c1] = a1
                    plsc.addupdate(stat_v.at[0, t], a0 + a1)
                    plsc.addupdate(stat_v.at[1, t], a0 * a0 + a1 * a1)

            plsc.parallel_loop(0, HJ2, unroll=2)(j_body)
            mus = []
            rs = []
            for t in range(C):
                s = _allreduce_sum(stat_v[0, t])
                q = _allreduce_sum(stat_v[1, t])
                mu = s * inv_h
                var = q * inv_h - mu * mu
                mus.append(mu)
                rs.append(_rsqrt_vec(var + jnp.float32(eps)))

            def j2_body(jj):
                col = pl.ds(jj * _L, _L)
                g = gam_v[col]
                b = bet_v[col]
                for t in range(C):
                    word_v[slot, t, col] = ((word_v[slot, t, col] - mus[t])
                                            * rs[t] * g + b)

            plsc.parallel_loop(0, H // _L, unroll=2)(j2_body)
            pltpu.sync_copy(word_v.at[slot],
                            out_hbm.at[wid, pl.ds(c * C, C)])

        for q in range(DEPTH - 1):
            issue(q, q)

        def group_body(p, carry):
            cbase = p * DEPTH
            issue(cbase + DEPTH - 1, DEPTH - 1)
            for q in range(DEPTH):
                drain(cbase + q, q)
                compute(cbase + q, q)
                if q < DEPTH - 1:
                    @pl.when(p < NCH // DEPTH - 1)
                    def _(q=q):
                        issue(cbase + DEPTH + q, q)

            return carry

        lax.fori_loop(0, NCH // DEPTH, group_body, 0)

    return pl.kernel(
        body,
        out_type=jax.ShapeDtypeStruct((B, S, H), jnp.float32),
        mesh=mesh,
        scratch_types=[
            pltpu.VMEM((8, S), jnp.int32),
            pltpu.VMEM((DEPTH, C, H), jnp.float32),
            pltpu.VMEM((DEPTH, 7, C, H2), jnp.int32),
            pltpu.VMEM((2, C, _L), jnp.float32),
            pltpu.VMEM((H,), jnp.float32),
            pltpu.VMEM((H,), jnp.float32),
            pltpu.SemaphoreType.DMA,
            pltpu.SemaphoreType.DMA,
            pltpu.SemaphoreType.DMA,
            pltpu.SemaphoreType.DMA,
        ],
    )


def _to_bf16_perm(t):
    v, h = t.shape
    b = t.astype(jnp.bfloat16)
    b = b.reshape(v, h // 32, 2, 16).transpose(0, 1, 3, 2)
    return lax.bitcast_convert_type(b, jnp.int32).reshape(v, h // 2)


def kernel(input_ids, bbox, token_type_ids, word_emb, x_emb, y_emb, h_emb,
           w_emb, pos_emb, tok_emb, gamma, beta):
    B, S = input_ids.shape
    H = word_emb.shape[1]
    C = 8
    npos = pos_emb.shape[0]
    pt = (tok_emb[:, None, :] + pos_emb[None, :, :]).reshape(-1, H)
    b0 = bbox[:, :, 0]
    b1 = bbox[:, :, 1]
    b2 = bbox[:, :, 2]
    b3 = bbox[:, :, 3]
    pos_ids = jnp.broadcast_to(jnp.arange(S, dtype=jnp.int32)[None, :],
                               (B, S))
    pt_ids = token_type_ids.astype(jnp.int32) * npos + pos_ids
    idx = jnp.stack([input_ids.astype(jnp.int32), b0, b1, b2, b3,
                     b3 - b1, b2 - b0, pt_ids], axis=1)
    k = _make_kernel(B, S, H, C, 1e-05)
    return k(idx, word_emb, _to_bf16_perm(x_emb), _to_bf16_perm(y_emb),
             _to_bf16_perm(h_emb), _to_bf16_perm(w_emb),
             _to_bf16_perm(pt), gamma, beta)
